# baseline (speedup 1.0000x reference)
import functools

import jax
import jax.numpy as jnp
from jax import lax
from jax.experimental import pallas as pl
from jax.experimental.pallas import tpu as pltpu
from jax.experimental.pallas import tpu_sc as plsc

_NUM_TASKS = 3
_PROMPT_LEN = 20
_HIDDEN = 4096
_BATCH = 1024

_NC = 2
_NS = 16
_NW = _NC * _NS
_NRANGE = 32
_RSZ = _BATCH // _NRANGE
_L = 16


def _sc_body(table_hbm, ids_hbm, out_hbm, ids_v, slice_v, sem):
    sid = lax.axis_index("s")
    cid = lax.axis_index("c")
    wid = sid * _NC + cid
    t_w = lax.rem(wid, _NUM_TASKS)
    q = lax.div(wid, _NUM_TASKS)
    gsz = jnp.where(t_w == _NUM_TASKS - 1, _NW // _NUM_TASKS,
                    _NW // _NUM_TASKS + 1).astype(jnp.int32)

    pltpu.sync_copy(table_hbm.at[pl.ds(t_w, 1)], slice_v)
    pltpu.sync_copy(ids_hbm, ids_v)

    nissued = jnp.int32(0)
    for rr in range(4):
        rid = q + rr * gsz
        valid = rid < _NRANGE
        rid_c = jnp.minimum(rid, _NRANGE - 1)
        for g in range(_RSZ // _L):
            b0 = rid_c * _RSZ + g * _L
            tvec = ids_v[pl.ds(b0, _L)]
            sel = (tvec == t_w) & valid
            seli = sel.astype(jnp.int32)
            for k in range(_L):
                cond = seli[k] != 0
                b = b0 + k

                @pl.when(cond)
                def _():
                    pltpu.async_copy(
                        slice_v, out_hbm.at[pl.ds(b, 1)], sem
                    )

            cnt = plsc.all_reduce_population_count(sel)
            nissued = nissued + cnt[0]

    def drain(e, carry):
        pltpu.make_async_copy(slice_v, out_hbm.at[pl.ds(0, 1)], sem).wait()
        return carry

    lax.fori_loop(0, nissued, drain, 0)


_sc_gather = functools.partial(
    pl.kernel,
    out_type=jax.ShapeDtypeStruct((_BATCH, _PROMPT_LEN, _HIDDEN), jnp.float32),
    mesh=plsc.VectorSubcoreMesh(core_axis_name="c", subcore_axis_name="s"),
    compiler_params=pltpu.CompilerParams(needs_layout_passes=False),
    scratch_types=[
        pltpu.VMEM((_BATCH,), jnp.int32),
        pltpu.VMEM((1, _PROMPT_LEN, _HIDDEN), jnp.float32),
        pltpu.SemaphoreType.DMA,
    ],
)(_sc_body)


def kernel(task_ids, prompt_embeddings):
    ids = task_ids.astype(jnp.int32)
    return _sc_gather(prompt_embeddings, ids)
